# baseline (speedup 1.0000x reference)
import jax
import jax.numpy as jnp
from jax.experimental import pallas as pl
from jax.experimental.pallas import tpu as pltpu

B, N, F_IN, F_OUT, K = 4, 2048, 256, 256, 3
BLK = 256
G = N // BLK
RBLK = (B - 1) * N // G
NSTEPS = 3 * G

_CD0 = (((0,), (0,)), ((), ()))


def _cheb_kernel(adj_ref, x0_ref, x0b_ref, xr_ref, w_ref, bias_ref,
                 out0_ref, outr_ref,
                 adj_bf, m1_s, m2_s, z1_s, t1_s, dinv_s, diag_s):
    s = pl.program_id(0)
    bias = bias_ref[:]

    @pl.when(s < G)
    def phase1():
        blk = adj_ref[:]
        rowsum = jnp.sum(blk, axis=1, keepdims=True)
        r = jax.lax.broadcasted_iota(jnp.int32, (BLK, N), 0) + s * BLK
        c = jax.lax.broadcasted_iota(jnp.int32, (BLK, N), 1)
        diag = jnp.sum(jnp.where(r == c, blk, 0.0), axis=1, keepdims=True)
        deg = rowsum - diag
        dinv = jnp.where(deg > 0, jax.lax.rsqrt(jnp.where(deg > 0, deg, 1.0)),
                         0.0)
        z1 = dinv * x0_ref[:]
        rows = pl.ds(s * BLK, BLK)
        blk_bf = blk.astype(jnp.bfloat16)
        adj_bf[rows, :] = blk_bf
        dinv_s[rows, :] = dinv
        diag_s[rows, :] = diag
        z1_s[rows, :] = z1
        acc = jax.lax.dot_general(blk_bf, z1.astype(jnp.bfloat16), _CD0,
                                  preferred_element_type=jnp.float32)

        @pl.when(s == 0)
        def _():
            m1_s[:] = acc

        @pl.when(s > 0)
        def _():
            m1_s[:] += acc

    @pl.when((s >= G) & (s < 2 * G))
    def phase2():
        j = s - G
        rows = pl.ds(j * BLK, BLK)
        dinv = dinv_s[rows, :]
        t1 = -dinv * (m1_s[rows, :] - diag_s[rows, :] * z1_s[rows, :])
        t1_s[rows, :] = t1
        z2 = dinv * t1
        acc = jax.lax.dot_general(adj_bf[rows, :], z2.astype(jnp.bfloat16),
                                  _CD0, preferred_element_type=jnp.float32)

        @pl.when(j == 0)
        def _():
            m2_s[:] = acc

        @pl.when(j > 0)
        def _():
            m2_s[:] += acc

    @pl.when(s >= 2 * G)
    def phase3():
        i = s - 2 * G
        rows = pl.ds(i * BLK, BLK)
        w0 = w_ref[0]
        w1 = w_ref[1].astype(jnp.bfloat16)
        w2 = w_ref[2]
        wc = (w0 - w2).astype(jnp.bfloat16)
        w2_bf = w2.astype(jnp.bfloat16)
        dinv = dinv_s[rows, :]
        t1 = t1_s[rows, :]
        t2 = -dinv * (m2_s[rows, :] - diag_s[rows, :] * (dinv * t1))
        out0_ref[:] = (
            jnp.dot(x0b_ref[:].astype(jnp.bfloat16), wc,
                    preferred_element_type=jnp.float32)
            + jnp.dot(t1.astype(jnp.bfloat16), w1,
                      preferred_element_type=jnp.float32)
            + jnp.dot((2.0 * t2).astype(jnp.bfloat16), w2_bf,
                      preferred_element_type=jnp.float32)
            + bias)
        outr_ref[:] = jnp.dot(xr_ref[:].astype(jnp.bfloat16), wc,
                              preferred_element_type=jnp.float32) + bias


def kernel(data, adj, W, bias):
    x0 = data[0]
    xr = data[1:].reshape((B - 1) * N, F_IN)

    def _p1(s):
        return (jnp.minimum(s, G - 1), 0)

    def _p3(s):
        return (jnp.maximum(jnp.minimum(s - 2 * G, G - 1), 0), 0)

    out0, outr = pl.pallas_call(
        _cheb_kernel,
        grid=(NSTEPS,),
        in_specs=[
            pl.BlockSpec((BLK, N), _p1),
            pl.BlockSpec((BLK, F_IN), _p1),
            pl.BlockSpec((BLK, F_IN), _p3),
            pl.BlockSpec((RBLK, F_IN), _p3),
            pl.BlockSpec((K, F_IN, F_OUT), lambda s: (0, 0, 0)),
            pl.BlockSpec((1, F_OUT), lambda s: (0, 0)),
        ],
        out_specs=[
            pl.BlockSpec((BLK, F_OUT), _p3),
            pl.BlockSpec((RBLK, F_OUT), _p3),
        ],
        out_shape=[
            jax.ShapeDtypeStruct((N, F_OUT), jnp.float32),
            jax.ShapeDtypeStruct(((B - 1) * N, F_OUT), jnp.float32),
        ],
        scratch_shapes=[
            pltpu.VMEM((N, N), jnp.bfloat16),
            pltpu.VMEM((N, F_IN), jnp.float32),
            pltpu.VMEM((N, F_IN), jnp.float32),
            pltpu.VMEM((N, F_IN), jnp.float32),
            pltpu.VMEM((N, F_IN), jnp.float32),
            pltpu.VMEM((N, 1), jnp.float32),
            pltpu.VMEM((N, 1), jnp.float32),
        ],
    )(adj, x0, x0, xr, W, bias.reshape(1, F_OUT))
    return jnp.concatenate([out0, outr], axis=0).reshape(B, N, F_OUT)


# speedup vs baseline: 2.0519x; 2.0519x over previous
import jax
import jax.numpy as jnp
from jax.experimental import pallas as pl
from jax.experimental.pallas import tpu as pltpu

B, N, F_IN, F_OUT, K = 4, 2048, 256, 256, 3
CHUNK = 256
NCH = N // CHUNK
NR = (B - 1) * N
XCH = NR // 4

_CD0 = (((0,), (0,)), ((), ()))


def _cheb_kernel(adj_hbm, data_hbm, w_ref, bias_ref, out_hbm,
                 adj_v, adj_bf, x0_v, z1_bf, dinv_v, diag_v,
                 xr_buf, outr_buf, out0_buf,
                 x0_sem, adj_sem, xr_sem, outw_sem):
    bias = bias_ref[:]
    w1_bf = w_ref[1].astype(jnp.bfloat16)
    w2_bf = w_ref[2].astype(jnp.bfloat16)
    wc_bf = (w_ref[0] - w_ref[2]).astype(jnp.bfloat16)

    x0_copy = pltpu.make_async_copy(data_hbm.at[pl.ds(0, N), :], x0_v, x0_sem)
    x0_copy.start()
    adj_copies = []
    for i in range(NCH):
        c = pltpu.make_async_copy(adj_hbm.at[pl.ds(i * CHUNK, CHUNK), :],
                                  adj_v.at[pl.ds(i * CHUNK, CHUNK), :],
                                  adj_sem.at[i])
        c.start()
        adj_copies.append(c)
    xr_copies = []
    for i in range(4):
        c = pltpu.make_async_copy(data_hbm.at[pl.ds(N + i * XCH, XCH), :],
                                  xr_buf.at[i], xr_sem.at[i])
        c.start()
        xr_copies.append(c)

    x0_copy.wait()
    for i in range(NCH):
        adj_copies[i].wait()
        sl = pl.ds(i * CHUNK, CHUNK)
        blk = adj_v[sl, :]
        rowsum = jnp.sum(blk, axis=1, keepdims=True)
        r = jax.lax.broadcasted_iota(jnp.int32, (CHUNK, N), 0) + i * CHUNK
        c = jax.lax.broadcasted_iota(jnp.int32, (CHUNK, N), 1)
        diag = jnp.sum(jnp.where(r == c, blk, 0.0), axis=1, keepdims=True)
        deg = rowsum - diag
        dinv = jnp.where(deg > 0, jax.lax.rsqrt(jnp.where(deg > 0, deg, 1.0)),
                         0.0)
        dinv_v[sl, :] = dinv
        diag_v[sl, :] = diag
        adj_bf[sl, :] = blk.astype(jnp.bfloat16)
        z1_bf[sl, :] = (dinv * x0_v[sl, :]).astype(jnp.bfloat16)

    out_copies = []
    for i in range(4):
        xr_copies[i].wait()
        o = jnp.dot(xr_buf[i].astype(jnp.bfloat16), wc_bf,
                    preferred_element_type=jnp.float32) + bias
        outr_buf[i] = o
        c = pltpu.make_async_copy(outr_buf.at[i],
                                  out_hbm.at[pl.ds(N + i * XCH, XCH), :],
                                  outw_sem.at[i])
        c.start()
        out_copies.append(c)

    dinv = dinv_v[:]
    diag = diag_v[:]
    m1 = jax.lax.dot_general(adj_bf[:], z1_bf[:], _CD0,
                             preferred_element_type=jnp.float32)
    z1f = dinv * x0_v[:]
    t1 = dinv * (diag * z1f - m1)
    z2 = dinv * t1
    m2 = jax.lax.dot_general(adj_bf[:], z2.astype(jnp.bfloat16), _CD0,
                             preferred_element_type=jnp.float32)
    t2 = dinv * (diag * z2 - m2)
    out0_buf[:] = (
        jnp.dot(x0_v[:].astype(jnp.bfloat16), wc_bf,
                preferred_element_type=jnp.float32)
        + jnp.dot(t1.astype(jnp.bfloat16), w1_bf,
                  preferred_element_type=jnp.float32)
        + jnp.dot((2.0 * t2).astype(jnp.bfloat16), w2_bf,
                  preferred_element_type=jnp.float32)
        + bias)
    c = pltpu.make_async_copy(out0_buf, out_hbm.at[pl.ds(0, N), :],
                              outw_sem.at[4])
    c.start()
    out_copies.append(c)
    for c in out_copies:
        c.wait()


def kernel(data, adj, W, bias):
    out = pl.pallas_call(
        _cheb_kernel,
        in_specs=[
            pl.BlockSpec(memory_space=pltpu.MemorySpace.HBM),
            pl.BlockSpec(memory_space=pltpu.MemorySpace.HBM),
            pl.BlockSpec(memory_space=pltpu.MemorySpace.VMEM),
            pl.BlockSpec(memory_space=pltpu.MemorySpace.VMEM),
        ],
        out_specs=pl.BlockSpec(memory_space=pltpu.MemorySpace.HBM),
        out_shape=jax.ShapeDtypeStruct((B * N, F_OUT), jnp.float32),
        scratch_shapes=[
            pltpu.VMEM((N, N), jnp.float32),
            pltpu.VMEM((N, N), jnp.bfloat16),
            pltpu.VMEM((N, F_IN), jnp.float32),
            pltpu.VMEM((N, F_IN), jnp.bfloat16),
            pltpu.VMEM((N, 1), jnp.float32),
            pltpu.VMEM((N, 1), jnp.float32),
            pltpu.VMEM((4, XCH, F_IN), jnp.float32),
            pltpu.VMEM((4, XCH, F_OUT), jnp.float32),
            pltpu.VMEM((N, F_OUT), jnp.float32),
            pltpu.SemaphoreType.DMA,
            pltpu.SemaphoreType.DMA((NCH,)),
            pltpu.SemaphoreType.DMA((4,)),
            pltpu.SemaphoreType.DMA((5,)),
        ],
    )(adj, data.reshape(B * N, F_IN), W, bias.reshape(1, F_OUT))
    return out.reshape(B, N, F_OUT)
